# Initial kernel scaffold; baseline (speedup 1.0000x reference)
import jax
import jax.numpy as jnp
from jax.experimental import pallas as pl


def kernel(node_feat, coord, edge_feat, conn_feat, params, od_edge_index, connect_edge_index):
    raise NotImplementedError("write your pallas kernel here")


# speedup vs baseline: 35.1985x; 35.1985x over previous
import functools

import jax
import jax.numpy as jnp
import numpy as np
from jax import lax
from jax.experimental import pallas as pl
from jax.experimental.pallas import tpu as pltpu
from jax.experimental.pallas import tpu_sc as plsc

N = 50000
DEG = 16
E = N * DEG
NH = 4
HF = 16
H = 64

BN = 2000
BE = 2000


def _b2(b):
    return b.reshape(1, -1)


def _node_dense_body(nf_ref, c8_ref, w1_ref, b1_ref, w2_ref, b2_ref,
                     wqh_ref, wqc_ref, bq_ref,
                     wkh_ref, wkc_ref, bk_ref,
                     wvh_ref, wvc_ref, bv_ref,
                     h0_ref, q_ref, k_ref, v_ref):
    nf = nf_ref[...]
    c8 = c8_ref[...]
    t = jnp.maximum(nf @ w1_ref[...] + b1_ref[...], 0.0)
    h0 = t @ w2_ref[...] + b2_ref[...]
    h0_ref[...] = h0
    q_ref[...] = h0 @ wqh_ref[...] + c8 @ wqc_ref[...] + bq_ref[...]
    k_ref[...] = h0 @ wkh_ref[...] + c8 @ wkc_ref[...] + bk_ref[...]
    v_ref[...] = h0 @ wvh_ref[...] + c8 @ wvc_ref[...] + bv_ref[...]


def _node_dense(nf, c8, p):
    grid = (N // BN,)
    row = lambda d: pl.BlockSpec((BN, d), lambda i: (i, 0))
    wsp = lambda a, b: pl.BlockSpec((a, b), lambda i: (0, 0))
    cv = p["conv1"]
    wq, wk, wv = cv["q"]["W"], cv["k"]["W"], cv["v"]["W"]
    pad26 = lambda w: jnp.pad(w[64:66], ((0, 6), (0, 0)))
    args = (nf, c8,
            p["pre1"]["W"], _b2(p["pre1"]["b"]), p["pre2"]["W"], _b2(p["pre2"]["b"]),
            wq[:64], pad26(wq), _b2(cv["q"]["b"]),
            wk[:64], pad26(wk), _b2(cv["k"]["b"]),
            wv[:64], pad26(wv), _b2(cv["v"]["b"]))
    in_specs = [row(128), row(8),
                wsp(128, 64), wsp(1, 64), wsp(64, 64), wsp(1, 64),
                wsp(64, 64), wsp(8, 64), wsp(1, 64),
                wsp(64, 64), wsp(8, 64), wsp(1, 64),
                wsp(64, 64), wsp(8, 64), wsp(1, 64)]
    outs = [jax.ShapeDtypeStruct((N, 64), jnp.float32)] * 4
    return pl.pallas_call(
        _node_dense_body, grid=grid, in_specs=in_specs,
        out_specs=[row(64)] * 4, out_shape=outs)(*args)


def _combine_body(a_const, wv_ref, z8_ref, e8_ref, xin_ref, c8_ref,
                  wo_ref, bo_ref, wfh_ref, wfc_ref, bf_ref, g_ref, bb_ref,
                  y_ref):
    z8 = z8_ref[...]
    mult = (1.0 / (z8 + a_const)) @ e8_ref[...]
    wvn = wv_ref[...] * mult
    o = wvn @ wo_ref[...] + bo_ref[...]
    h = xin_ref[...] @ wfh_ref[...] + c8_ref[...] @ wfc_ref[...] + bf_ref[...] + o
    m = jnp.mean(h, -1, keepdims=True)
    var = jnp.mean((h - m) ** 2, -1, keepdims=True)
    ln = (h - m) * jax.lax.rsqrt(var + 1e-5) * g_ref[...] + bb_ref[...]
    y = h + ln
    y_ref[...] = jnp.where(y >= 0, y, 0.01 * y)


def _combine(wv, z8, xin, c8, mp, wfh, wfc, a_const):
    e8 = jnp.zeros((8, 64), jnp.float32).at[
        jnp.repeat(jnp.arange(4), 16), jnp.arange(64)].set(1.0)
    grid = (N // BN,)
    row = lambda d: pl.BlockSpec((BN, d), lambda i: (i, 0))
    wsp = lambda a, b: pl.BlockSpec((a, b), lambda i: (0, 0))
    args = (wv, z8, e8, xin, c8,
            mp["o"]["W"], _b2(mp["o"]["b"]), wfh, wfc, _b2(mp["f"]["b"]),
            _b2(mp["ln_g"]), _b2(mp["ln_b"]))
    in_specs = [row(64), row(8), wsp(8, 64), row(64), row(8),
                wsp(64, 64), wsp(1, 64), wsp(64, 64), wsp(8, 64), wsp(1, 64),
                wsp(1, 64), wsp(1, 64)]
    return pl.pallas_call(
        functools.partial(_combine_body, a_const), grid=grid,
        in_specs=in_specs, out_specs=row(64),
        out_shape=jax.ShapeDtypeStruct((N, 64), jnp.float32))(*args)


def _qkv_body(y_ref, wq_ref, bq_ref, wk_ref, bk_ref, wv_ref, bv_ref,
              q_ref, k_ref, v_ref):
    y = y_ref[...]
    q_ref[...] = y @ wq_ref[...] + bq_ref[...]
    k_ref[...] = y @ wk_ref[...] + bk_ref[...]
    v_ref[...] = y @ wv_ref[...] + bv_ref[...]


def _qkv(y, mp):
    grid = (N // BN,)
    row = lambda d: pl.BlockSpec((BN, d), lambda i: (i, 0))
    wsp = lambda a, b: pl.BlockSpec((a, b), lambda i: (0, 0))
    args = (y, mp["q"]["W"], _b2(mp["q"]["b"]), mp["k"]["W"], _b2(mp["k"]["b"]),
            mp["v"]["W"], _b2(mp["v"]["b"]))
    in_specs = [row(64)] + [wsp(64, 64), wsp(1, 64)] * 3
    outs = [jax.ShapeDtypeStruct((N, 64), jnp.float32)] * 3
    return pl.pallas_call(
        _qkv_body, grid=grid, in_specs=in_specs,
        out_specs=[row(64)] * 3, out_shape=outs)(*args)


def _edge_mlp_body(gcs_ref, gcd_ref, ef8_ref, w1a_ref, w1b_ref, w1e_ref, b1_ref,
                   wm0_ref, bm0_ref, wm1_ref, bm1_ref, wm2_ref, bm2_ref,
                   w2_ref, b2_ref, out_ref):
    r = jnp.maximum(gcs_ref[...] @ w1a_ref[...] + gcd_ref[...] @ w1b_ref[...]
                    + ef8_ref[...] @ w1e_ref[...] + b1_ref[...], 0.0)
    r = jnp.maximum(r @ wm0_ref[...] + bm0_ref[...], 0.0)
    r = jnp.maximum(r @ wm1_ref[...] + bm1_ref[...], 0.0)
    r = jnp.maximum(r @ wm2_ref[...] + bm2_ref[...], 0.0)
    out_ref[...] = r @ w2_ref[...] + b2_ref[...]


def _edge_mlp(gcs, gcd, ef8, p):
    w134 = p["reg1"]["W"]
    w1a = jnp.concatenate([w134[0:64], w134[128:130],
                           jnp.zeros((14, 128), jnp.float32)], 0)
    w1b = jnp.concatenate([w134[64:128], w134[130:132],
                           jnp.zeros((14, 128), jnp.float32)], 0)
    w1e = jnp.pad(w134[132:134], ((0, 6), (0, 0)))
    grid = (E // BE,)
    row = lambda d: pl.BlockSpec((BE, d), lambda i: (i, 0))
    wsp = lambda a, b: pl.BlockSpec((a, b), lambda i: (0, 0))
    args = [gcs, gcd, ef8, w1a, w1b, w1e, _b2(p["reg1"]["b"])]
    in_specs = [row(80), row(80), row(8),
                wsp(80, 128), wsp(80, 128), wsp(8, 128), wsp(1, 128)]
    for lp in p["regm"]:
        args += [lp["W"], _b2(lp["b"])]
        in_specs += [wsp(128, 128), wsp(1, 128)]
    args += [p["reg2"]["W"], _b2(p["reg2"]["b"])]
    in_specs += [wsp(128, 1), wsp(1, 1)]
    return pl.pallas_call(
        _edge_mlp_body, grid=grid, in_specs=in_specs, out_specs=row(1),
        out_shape=jax.ShapeDtypeStruct((E, 1), jnp.float32))(*args)


_CE = 128
_NCHUNK = E // _CE
_NZF = N // _CE
_AW = 32


def _make_edge_sc(scale):
    mesh = plsc.VectorSubcoreMesh(core_axis_name="c", subcore_axis_name="s")

    @functools.partial(
        pl.kernel, mesh=mesh,
        compiler_params=pltpu.CompilerParams(use_tc_tiling_on_sc=False,
                                             needs_layout_passes=False),
        out_type=[jax.ShapeDtypeStruct((2 * N, _AW), jnp.float32),
                  jax.ShapeDtypeStruct((2 * E, 8), jnp.float32)],
        scratch_types=[
            pltpu.VMEM((4, _CE), jnp.int32),
            pltpu.VMEM((4, _CE), jnp.int32),
            pltpu.VMEM((_CE, 64), jnp.float32),
            pltpu.VMEM((_CE, 64), jnp.float32),
            pltpu.VMEM((_CE, _AW), jnp.float32),
            pltpu.VMEM((_CE, _AW), jnp.float32),
            pltpu.VMEM((_CE, 8), jnp.float32),
            pltpu.VMEM((_CE,), jnp.int32),
            pltpu.VMEM((_CE,), jnp.float32),
            pltpu.VMEM_SHARED((N, _AW), jnp.float32),
            pltpu.SemaphoreType.DMA,
            pltpu.SemaphoreType.DMA,
            pltpu.SemaphoreType.DMA,
            pltpu.SemaphoreType.DMA,
        ],
    )
    def body(kqt, vt, edata, acc_hbm, sco_hbm,
             eb0, eb1, kq0, kq1, or0, or1, srows, dstc, connc,
             acc_sh, isem, gsem, ssem, wsem):
        c = lax.axis_index("c")
        s = lax.axis_index("s")
        cN = c * N
        iota = lax.iota(jnp.int32, 16)
        zero16 = jnp.zeros((16,), jnp.float32)
        ebufs = (eb0, eb1)
        kqbufs = (kq0, kq1)
        obufs = (or0, or1)

        def zo(i, _):
            fl = i * 16 + iota
            plsc.store_scatter(or0, [fl // _AW, fl % _AW], zero16)
            return 0
        lax.fori_loop(0, _CE * _AW // 16, zo, 0)

        def zs(i, _):
            fl = i * 16 + iota
            plsc.store_scatter(srows, [fl // 8, fl % 8], zero16)
            return 0
        lax.fori_loop(0, _CE * 8 // 16, zs, 0)

        def zinit(i, _):
            cid = i * 16 + s

            @pl.when(cid < _NZF)
            def _():
                pltpu.sync_copy(or0, acc_sh.at[pl.ds(cid * _CE, _CE)])
            return 0
        lax.fori_loop(0, (_NZF + 15) // 16, zinit, 0)

        @pl.when(s == 0)
        def _():
            pltpu.sync_copy(or0.at[pl.ds(0, N - _NZF * _CE)],
                            acc_sh.at[pl.ds(_NZF * _CE, N - _NZF * _CE)])
        plsc.subcore_barrier()

        cnt = jnp.where(s < 10, 391, 390).astype(jnp.int32)
        start = s * 390 + jnp.minimum(s, 10)

        def fetch(j, b):
            pltpu.make_async_copy(edata.at[pl.ds(0, 4)], ebufs[b], isem).wait()
            for t in range(_CE // 16):
                sl = pl.ds(t * 16, 16)
                ebufs[b][3, sl] = ebufs[b][0, sl] + cN

            @pl.when(j >= 2)
            def _():
                pltpu.make_async_copy(
                    or0, acc_sh.at[pl.ds(0, _CE)], ssem).wait()
                pltpu.make_async_copy(
                    srows, sco_hbm.at[pl.ds(0, _CE)], wsem).wait()
            pltpu.async_copy(kqt.at[ebufs[b].at[3]], kqbufs[b], gsem)
            pltpu.async_copy(vt.at[ebufs[b].at[3]], obufs[b], gsem)

        def consume(j, b, cnt):
            o = 1 - b
            pltpu.make_async_copy(kqt.at[pl.ds(0, _CE)], kqbufs[o], gsem).wait()
            pltpu.make_async_copy(vt.at[pl.ds(0, _CE)], obufs[o], gsem).wait()

            @pl.when(j == cnt)
            def _():
                pltpu.make_async_copy(
                    or0, acc_sh.at[pl.ds(0, _CE)], ssem).wait()
                pltpu.make_async_copy(
                    srows, sco_hbm.at[pl.ds(0, _CE)], wsem).wait()
            for t in range(_CE // 16):
                sl = pl.ds(t * 16, 16)
                dstc[sl] = ebufs[o][1, sl]
                connc[sl] = plsc.bitcast(ebufs[o][2, sl], jnp.float32)

            @pl.when(j + 1 < cnt)
            def _():
                nxt = start + j + 1
                pltpu.async_copy(edata.at[pl.ds(nxt * 4, 4)], ebufs[o], isem)

            def grp(g, _):
                ev = g * 16 + iota
                cn = connc[pl.ds(g * 16, 16)]
                for h in range(2):
                    sacc = zero16
                    for t in range(16):
                        rot = jnp.bitwise_and(t + iota, 15)
                        sacc = sacc + (
                            plsc.load_gather(kqbufs[o], [ev, h * 16 + rot])
                            * plsc.load_gather(kqbufs[o],
                                               [ev, 32 + h * 16 + rot]))
                    sc = jnp.clip(sacc * cn * scale, -5.0, 5.0)
                    sc = jnp.exp(sc)
                    for t in range(16):
                        rot = jnp.bitwise_and(t + iota, 15)
                        colv = h * 16 + rot
                        vv = plsc.load_gather(obufs[o], [ev, colv])
                        plsc.store_scatter(obufs[o], [ev, colv], vv * sc)
                    plsc.store_scatter(
                        srows, [ev, jnp.full((16,), h, jnp.int32)], sc)
                return 0
            lax.fori_loop(0, _CE // 16, grp, 0)
            pltpu.async_copy(obufs[o], acc_sh.at[dstc], ssem, add=True)
            pltpu.async_copy(
                srows, sco_hbm.at[pl.ds(c * E + (start + j - 1) * _CE, _CE)],
                wsem)

        pltpu.async_copy(edata.at[pl.ds(start * 4, 4)], ebufs[0], isem)
        pltpu.async_copy(edata.at[pl.ds((start + 1) * 4, 4)], ebufs[1], isem)

        def pair_body(jj, _):
            for b in range(2):
                j = jj * 2 + b

                @pl.when(j < cnt)
                def _():
                    fetch(j, b)

                @pl.when(jnp.logical_and(j >= 1, j <= cnt))
                def _():
                    consume(j, b, cnt)
            return 0
        lax.fori_loop(0, (391 + 2) // 2, pair_body, 0)

        pltpu.make_async_copy(or0, acc_sh.at[pl.ds(0, _CE)], ssem).wait()
        pltpu.make_async_copy(srows, sco_hbm.at[pl.ds(0, _CE)], wsem).wait()
        plsc.subcore_barrier()

        def cpout(i, _):
            cid = i * 16 + s

            @pl.when(cid < _NZF)
            def _():
                pltpu.sync_copy(acc_sh.at[pl.ds(cid * _CE, _CE)], or0)
                pltpu.sync_copy(or0, acc_hbm.at[pl.ds(cN + cid * _CE, _CE)])
            return 0
        lax.fori_loop(0, (_NZF + 15) // 16, cpout, 0)

        @pl.when(s == 0)
        def _():
            tail = N - _NZF * _CE
            pltpu.sync_copy(acc_sh.at[pl.ds(_NZF * _CE, tail)],
                            or0.at[pl.ds(0, tail)])
            pltpu.sync_copy(or0.at[pl.ds(0, tail)],
                            acc_hbm.at[pl.ds(cN + _NZF * _CE, tail)])

    return body


def _z_sc():
    mesh = plsc.VectorSubcoreMesh(core_axis_name="c", subcore_axis_name="s")

    @functools.partial(
        pl.kernel, mesh=mesh,
        compiler_params=pltpu.CompilerParams(use_tc_tiling_on_sc=False,
                                             needs_layout_passes=False),
        out_type=jax.ShapeDtypeStruct((2 * N, 8), jnp.float32),
        scratch_types=[
            pltpu.VMEM((_CE,), jnp.int32),
            pltpu.VMEM((_CE, 8), jnp.float32),
            pltpu.VMEM_SHARED((N, 8), jnp.float32),
        ],
    )
    def body(scol, dstl, z_hbm, dst_v, srows, accz):
        c = lax.axis_index("c")
        s = lax.axis_index("s")
        cN = c * N
        iota = lax.iota(jnp.int32, 16)
        zero16 = jnp.zeros((16,), jnp.float32)

        def zs(i, _):
            fl = i * 16 + iota
            plsc.store_scatter(srows, [fl // 8, fl % 8], zero16)
            return 0
        lax.fori_loop(0, _CE * 8 // 16, zs, 0)

        def zinit(i, _):
            cid = i * 16 + s

            @pl.when(cid < _NZF)
            def _():
                pltpu.sync_copy(srows, accz.at[pl.ds(cid * _CE, _CE)])
            return 0
        lax.fori_loop(0, (_NZF + 15) // 16, zinit, 0)

        @pl.when(s == 0)
        def _():
            pltpu.sync_copy(srows.at[pl.ds(0, N - _NZF * _CE)],
                            accz.at[pl.ds(_NZF * _CE, N - _NZF * _CE)])
        plsc.subcore_barrier()

        def chunk_body(i, _):
            cid = i * 16 + s

            @pl.when(cid < _NCHUNK)
            def _():
                base = cid * _CE
                pltpu.sync_copy(dstl.at[pl.ds(base, _CE)], dst_v)
                pltpu.sync_copy(scol.at[pl.ds(c * E + base, _CE)], srows)
                pltpu.sync_copy(srows, accz.at[dst_v], add=True)
            return 0
        lax.fori_loop(0, (_NCHUNK + 15) // 16, chunk_body, 0)
        plsc.subcore_barrier()

        def cpout(i, _):
            cid = i * 16 + s

            @pl.when(cid < _NZF)
            def _():
                pltpu.sync_copy(accz.at[pl.ds(cid * _CE, _CE)], srows)
                pltpu.sync_copy(srows, z_hbm.at[pl.ds(cN + cid * _CE, _CE)])
            return 0
        lax.fori_loop(0, (_NZF + 15) // 16, cpout, 0)

        @pl.when(s == 0)
        def _():
            tail = N - _NZF * _CE
            pltpu.sync_copy(accz.at[pl.ds(_NZF * _CE, tail)],
                            srows.at[pl.ds(0, tail)])
            pltpu.sync_copy(srows.at[pl.ds(0, tail)],
                            z_hbm.at[pl.ds(cN + _NZF * _CE, tail)])

    return body


def _msg_sc(q, k, v, src, dst, conn, in_feats):
    kq = jnp.concatenate([k.reshape(N, 2, 32), q.reshape(N, 2, 32)], 2)
    kqt = kq.transpose(1, 0, 2).reshape(2 * N, 64)
    vt = v.reshape(N, 2, 32).transpose(1, 0, 2).reshape(2 * N, 32)
    connb = jax.lax.bitcast_convert_type(conn, jnp.int32)
    ed = jnp.stack([src, dst, connb, jnp.zeros_like(src)], 0)
    edata = ed.reshape(4, _NCHUNK, _CE).transpose(1, 0, 2).reshape(
        _NCHUNK * 4, _CE)
    fn = _make_edge_sc(float(1.0 / np.sqrt(in_feats)))
    acc, scores = fn(kqt, vt, edata)
    zout = _z_sc()(scores, dst)
    wv = jnp.concatenate([acc[:N, :32], acc[N:, :32]], 1)
    z = jnp.concatenate([zout[:N, 0:2], zout[N:, 0:2]], 1)
    return wv, z


def _gather_sc(table, cs, cd):
    mesh = plsc.VectorSubcoreMesh(core_axis_name="c", subcore_axis_name="s")

    @functools.partial(
        pl.kernel, mesh=mesh,
        compiler_params=pltpu.CompilerParams(use_tc_tiling_on_sc=False,
                                             needs_layout_passes=False),
        out_type=[jax.ShapeDtypeStruct((E, 80), jnp.float32)] * 2,
        scratch_types=[
            pltpu.VMEM((_CE,), jnp.int32),
            pltpu.VMEM((_CE,), jnp.int32),
            pltpu.VMEM((_CE, 80), jnp.float32),
            pltpu.VMEM((_CE, 80), jnp.float32),
            pltpu.SemaphoreType.DMA,
        ],
    )
    def body(tab, csl, cdl, gcs_hbm, gcd_hbm, cs_v, cd_v, rows_a, rows_b, sem):
        c = lax.axis_index("c")
        s = lax.axis_index("s")
        w = s * 2 + c

        def chunk_body(i, _):
            cid = i * 32 + w

            @pl.when(cid < _NCHUNK)
            def _():
                base = cid * _CE
                pltpu.sync_copy(csl.at[pl.ds(base, _CE)], cs_v)
                pltpu.sync_copy(cdl.at[pl.ds(base, _CE)], cd_v)
                pltpu.async_copy(tab.at[cs_v], rows_a, sem).wait()
                pltpu.sync_copy(rows_a, gcs_hbm.at[pl.ds(base, _CE)])
                pltpu.async_copy(tab.at[cd_v], rows_b, sem).wait()
                pltpu.sync_copy(rows_b, gcd_hbm.at[pl.ds(base, _CE)])
            return 0
        lax.fori_loop(0, (_NCHUNK + 31) // 32, chunk_body, 0)

    return body(table, cs, cd)


def kernel(node_feat, coord, edge_feat, conn_feat, params, od_edge_index,
           connect_edge_index):
    p = params
    c8 = jnp.pad(coord, ((0, 0), (0, 6)))
    zeros86 = jnp.zeros((8, 64), jnp.float32)
    os_, od_ = od_edge_index[0], od_edge_index[1]
    cs_, cd_ = connect_edge_index[0], connect_edge_index[1]

    h0, q1, k1, v1 = _node_dense(node_feat, c8, p)

    ones_e = jnp.ones((E,), jnp.float32)
    conn_e = conn_feat[:, 0]

    wv, z = _msg_sc(q1, k1, v1, os_, od_, ones_e, 66.0)
    z8 = jnp.pad(z, ((0, 0), (0, 4)), constant_values=1.0)
    wf = p["conv1"]["f"]["W"]
    y1 = _combine(wv, z8, h0, c8, p["conv1"], wf[:64],
                  jnp.pad(wf[64:66], ((0, 6), (0, 0))), 1.0)

    q2, k2, v2 = _qkv(y1, p["cc1"])
    wv, z = _msg_sc(q2, k2, v2, cs_, cd_, conn_e, 64.0)
    z8 = jnp.pad(z, ((0, 0), (0, 4)), constant_values=1.0)
    y2 = _combine(wv, z8, y1, c8, p["cc1"], p["cc1"]["f"]["W"], zeros86, 0.0)

    q3, k3, v3 = _qkv(y2, p["cc2"])
    wv, z = _msg_sc(q3, k3, v3, cs_, cd_, conn_e, 64.0)
    z8 = jnp.pad(z, ((0, 0), (0, 4)), constant_values=1.0)
    y3 = _combine(wv, z8, y2, c8, p["cc2"], p["cc2"]["f"]["W"], zeros86, 0.0)

    hc80 = jnp.pad(jnp.concatenate([y3, coord], 1), ((0, 0), (0, 14)))
    gcs, gcd = _gather_sc(hc80, cs_, cd_)
    ef8 = jnp.pad(edge_feat, ((0, 0), (0, 6)))
    return _edge_mlp(gcs, gcd, ef8, p)


# speedup vs baseline: 36.3399x; 1.0324x over previous
import functools

import jax
import jax.numpy as jnp
import numpy as np
from jax import lax
from jax.experimental import pallas as pl
from jax.experimental.pallas import tpu as pltpu
from jax.experimental.pallas import tpu_sc as plsc

N = 50000
DEG = 16
E = N * DEG
NH = 4
HF = 16
H = 64

BN = 2000
BE = 2000


def _b2(b):
    return b.reshape(1, -1)


def _kqv_weights(wq, bq, wk, bk, wv, bv):
    ws, bs = [], []
    for c in (0, 1):
        sl = slice(32 * c, 32 * c + 32)
        ws.append(jnp.concatenate([wk[:, sl], wq[:, sl]], 1))
        bs.append(jnp.concatenate([bk[sl], bq[sl]]).reshape(1, -1))
    for c in (0, 1):
        sl = slice(32 * c, 32 * c + 32)
        ws.append(wv[:, sl])
        bs.append(bv[sl].reshape(1, -1))
    return ws, bs


def _node_dense_body(nf_ref, c8_ref, w1_ref, b1_ref, w2_ref, b2_ref,
                     wa_ref, wa2_ref, ba_ref, wb_ref, wb2_ref, bb_ref,
                     wc_ref, wc2_ref, bc_ref, wd_ref, wd2_ref, bd_ref,
                     h0_ref, kq0_ref, kq1_ref, v0_ref, v1_ref):
    nf = nf_ref[...]
    c8 = c8_ref[...]
    t = jnp.maximum(nf @ w1_ref[...] + b1_ref[...], 0.0)
    h0 = t @ w2_ref[...] + b2_ref[...]
    h0_ref[...] = h0
    kq0_ref[...] = h0 @ wa_ref[...] + c8 @ wa2_ref[...] + ba_ref[...]
    kq1_ref[...] = h0 @ wb_ref[...] + c8 @ wb2_ref[...] + bb_ref[...]
    v0_ref[...] = h0 @ wc_ref[...] + c8 @ wc2_ref[...] + bc_ref[...]
    v1_ref[...] = h0 @ wd_ref[...] + c8 @ wd2_ref[...] + bd_ref[...]


def _node_dense(nf, c8, p):
    grid = (N // BN,)
    row = lambda d: pl.BlockSpec((BN, d), lambda i: (i, 0))
    wsp = lambda a, b: pl.BlockSpec((a, b), lambda i: (0, 0))
    cv = p["conv1"]
    ws, bs = _kqv_weights(cv["q"]["W"], cv["q"]["b"], cv["k"]["W"],
                          cv["k"]["b"], cv["v"]["W"], cv["v"]["b"])
    pad8 = lambda w: jnp.pad(w[64:66], ((0, 6), (0, 0)))
    args = [nf, c8,
            p["pre1"]["W"], _b2(p["pre1"]["b"]), p["pre2"]["W"],
            _b2(p["pre2"]["b"])]
    in_specs = [row(128), row(8),
                wsp(128, 64), wsp(1, 64), wsp(64, 64), wsp(1, 64)]
    for w, b in zip(ws, bs):
        d = w.shape[1]
        args += [w[:64], pad8(w), b]
        in_specs += [wsp(64, d), wsp(8, d), wsp(1, d)]
    outs = [jax.ShapeDtypeStruct((N, 64), jnp.float32)] * 3 + \
           [jax.ShapeDtypeStruct((N, 32), jnp.float32)] * 2
    return pl.pallas_call(
        _node_dense_body, grid=grid, in_specs=in_specs,
        out_specs=[row(64)] * 3 + [row(32)] * 2, out_shape=outs)(*args)


def _combine_body(a_const, wv_ref, z8_ref, e8_ref, xin_ref, c8_ref,
                  wo_ref, bo_ref, wfh_ref, wfc_ref, bf_ref, g_ref, bb_ref,
                  y_ref):
    z8 = z8_ref[...]
    mult = (1.0 / (z8 + a_const)) @ e8_ref[...]
    wvn = wv_ref[...] * mult
    o = wvn @ wo_ref[...] + bo_ref[...]
    h = xin_ref[...] @ wfh_ref[...] + c8_ref[...] @ wfc_ref[...] + bf_ref[...] + o
    m = jnp.mean(h, -1, keepdims=True)
    var = jnp.mean((h - m) ** 2, -1, keepdims=True)
    ln = (h - m) * jax.lax.rsqrt(var + 1e-5) * g_ref[...] + bb_ref[...]
    y = h + ln
    y_ref[...] = jnp.where(y >= 0, y, 0.01 * y)


def _combine(wv, z8, xin, c8, mp, wfh, wfc, a_const):
    e8 = jnp.zeros((8, 64), jnp.float32).at[
        jnp.repeat(jnp.arange(4), 16), jnp.arange(64)].set(1.0)
    grid = (N // BN,)
    row = lambda d: pl.BlockSpec((BN, d), lambda i: (i, 0))
    wsp = lambda a, b: pl.BlockSpec((a, b), lambda i: (0, 0))
    args = (wv, z8, e8, xin, c8,
            mp["o"]["W"], _b2(mp["o"]["b"]), wfh, wfc, _b2(mp["f"]["b"]),
            _b2(mp["ln_g"]), _b2(mp["ln_b"]))
    in_specs = [row(64), row(8), wsp(8, 64), row(64), row(8),
                wsp(64, 64), wsp(1, 64), wsp(64, 64), wsp(8, 64), wsp(1, 64),
                wsp(1, 64), wsp(1, 64)]
    return pl.pallas_call(
        functools.partial(_combine_body, a_const), grid=grid,
        in_specs=in_specs, out_specs=row(64),
        out_shape=jax.ShapeDtypeStruct((N, 64), jnp.float32))(*args)


def _qkv_body(y_ref, wa_ref, ba_ref, wb_ref, bb_ref, wc_ref, bc_ref,
              wd_ref, bd_ref, kq0_ref, kq1_ref, v0_ref, v1_ref):
    y = y_ref[...]
    kq0_ref[...] = y @ wa_ref[...] + ba_ref[...]
    kq1_ref[...] = y @ wb_ref[...] + bb_ref[...]
    v0_ref[...] = y @ wc_ref[...] + bc_ref[...]
    v1_ref[...] = y @ wd_ref[...] + bd_ref[...]


def _qkv(y, mp):
    grid = (N // BN,)
    row = lambda d: pl.BlockSpec((BN, d), lambda i: (i, 0))
    wsp = lambda a, b: pl.BlockSpec((a, b), lambda i: (0, 0))
    ws, bs = _kqv_weights(mp["q"]["W"], mp["q"]["b"], mp["k"]["W"],
                          mp["k"]["b"], mp["v"]["W"], mp["v"]["b"])
    args = [y]
    in_specs = [row(64)]
    for w, b in zip(ws, bs):
        args += [w, b]
        in_specs += [wsp(64, w.shape[1]), wsp(1, w.shape[1])]
    outs = [jax.ShapeDtypeStruct((N, 64), jnp.float32)] * 2 + \
           [jax.ShapeDtypeStruct((N, 32), jnp.float32)] * 2
    return pl.pallas_call(
        _qkv_body, grid=grid, in_specs=in_specs,
        out_specs=[row(64)] * 2 + [row(32)] * 2, out_shape=outs)(*args)


def _edge_mlp_body(gcs_ref, gcd_ref, ef8_ref, w1a_ref, w1b_ref, w1e_ref, b1_ref,
                   wm0_ref, bm0_ref, wm1_ref, bm1_ref, wm2_ref, bm2_ref,
                   w2_ref, b2_ref, out_ref):
    r = jnp.maximum(gcs_ref[...] @ w1a_ref[...] + gcd_ref[...] @ w1b_ref[...]
                    + ef8_ref[...] @ w1e_ref[...] + b1_ref[...], 0.0)
    r = jnp.maximum(r @ wm0_ref[...] + bm0_ref[...], 0.0)
    r = jnp.maximum(r @ wm1_ref[...] + bm1_ref[...], 0.0)
    r = jnp.maximum(r @ wm2_ref[...] + bm2_ref[...], 0.0)
    out_ref[...] = r @ w2_ref[...] + b2_ref[...]


def _edge_mlp(gcs, gcd, ef8, p):
    w134 = p["reg1"]["W"]
    w1a = jnp.concatenate([w134[0:64], w134[128:130],
                           jnp.zeros((14, 128), jnp.float32)], 0)
    w1b = jnp.concatenate([w134[64:128], w134[130:132],
                           jnp.zeros((14, 128), jnp.float32)], 0)
    w1e = jnp.pad(w134[132:134], ((0, 6), (0, 0)))
    grid = (E // BE,)
    row = lambda d: pl.BlockSpec((BE, d), lambda i: (i, 0))
    wsp = lambda a, b: pl.BlockSpec((a, b), lambda i: (0, 0))
    args = [gcs, gcd, ef8, w1a, w1b, w1e, _b2(p["reg1"]["b"])]
    in_specs = [row(80), row(80), row(8),
                wsp(80, 128), wsp(80, 128), wsp(8, 128), wsp(1, 128)]
    for lp in p["regm"]:
        args += [lp["W"], _b2(lp["b"])]
        in_specs += [wsp(128, 128), wsp(1, 128)]
    args += [p["reg2"]["W"], _b2(p["reg2"]["b"])]
    in_specs += [wsp(128, 1), wsp(1, 1)]
    return pl.pallas_call(
        _edge_mlp_body, grid=grid, in_specs=in_specs, out_specs=row(1),
        out_shape=jax.ShapeDtypeStruct((E, 1), jnp.float32))(*args)


_CE = 128
_NCHUNK = E // _CE
_NZF = N // _CE
_AW = 32


def _make_edge_sc(scale):
    mesh = plsc.VectorSubcoreMesh(core_axis_name="c", subcore_axis_name="s")

    @functools.partial(
        pl.kernel, mesh=mesh,
        compiler_params=pltpu.CompilerParams(use_tc_tiling_on_sc=False,
                                             needs_layout_passes=False),
        out_type=[jax.ShapeDtypeStruct((2 * N, _AW), jnp.float32),
                  jax.ShapeDtypeStruct((2 * E, 8), jnp.float32)],
        scratch_types=[
            pltpu.VMEM((4, _CE), jnp.int32),
            pltpu.VMEM((4, _CE), jnp.int32),
            pltpu.VMEM((_CE, 64), jnp.float32),
            pltpu.VMEM((_CE, 64), jnp.float32),
            pltpu.VMEM((_CE, _AW), jnp.float32),
            pltpu.VMEM((_CE, _AW), jnp.float32),
            pltpu.VMEM((_CE, 8), jnp.float32),
            pltpu.VMEM((_CE,), jnp.int32),
            pltpu.VMEM((_CE,), jnp.float32),
            pltpu.VMEM_SHARED((N, _AW), jnp.float32),
            pltpu.SemaphoreType.DMA,
            pltpu.SemaphoreType.DMA,
            pltpu.SemaphoreType.DMA,
            pltpu.SemaphoreType.DMA,
        ],
    )
    def body(kqt0, kqt1, vt0, vt1, edata, acc_hbm, sco_hbm,
             eb0, eb1, kq0, kq1, or0, or1, srows, dstc, connc,
             acc_sh, isem, gsem, ssem, wsem):
        c = lax.axis_index("c")
        s = lax.axis_index("s")
        cN = c * N
        iota = lax.iota(jnp.int32, 16)
        zero16 = jnp.zeros((16,), jnp.float32)
        ebufs = (eb0, eb1)
        kqbufs = (kq0, kq1)
        obufs = (or0, or1)

        def zo(i, _):
            fl = i * 16 + iota
            plsc.store_scatter(or0, [fl // _AW, fl % _AW], zero16)
            return 0
        lax.fori_loop(0, _CE * _AW // 16, zo, 0)

        def zs(i, _):
            fl = i * 16 + iota
            plsc.store_scatter(srows, [fl // 8, fl % 8], zero16)
            return 0
        lax.fori_loop(0, _CE * 8 // 16, zs, 0)

        def zinit(i, _):
            cid = i * 16 + s

            @pl.when(cid < _NZF)
            def _():
                pltpu.sync_copy(or0, acc_sh.at[pl.ds(cid * _CE, _CE)])
            return 0
        lax.fori_loop(0, (_NZF + 15) // 16, zinit, 0)

        @pl.when(s == 0)
        def _():
            pltpu.sync_copy(or0.at[pl.ds(0, N - _NZF * _CE)],
                            acc_sh.at[pl.ds(_NZF * _CE, N - _NZF * _CE)])
        plsc.subcore_barrier()

        cnt = jnp.where(s < 10, 391, 390).astype(jnp.int32)
        start = s * 390 + jnp.minimum(s, 10)

        def fetch(j, b):
            pltpu.make_async_copy(edata.at[pl.ds(0, 4)], ebufs[b], isem).wait()

            @pl.when(j >= 2)
            def _():
                pltpu.make_async_copy(
                    or0, acc_sh.at[pl.ds(0, _CE)], ssem).wait()
                pltpu.make_async_copy(
                    srows, sco_hbm.at[pl.ds(0, _CE)], wsem).wait()

            @pl.when(c == 0)
            def _():
                pltpu.async_copy(kqt0.at[ebufs[b].at[0]], kqbufs[b], gsem)
                pltpu.async_copy(vt0.at[ebufs[b].at[0]], obufs[b], gsem)

            @pl.when(c == 1)
            def _():
                pltpu.async_copy(kqt1.at[ebufs[b].at[0]], kqbufs[b], gsem)
                pltpu.async_copy(vt1.at[ebufs[b].at[0]], obufs[b], gsem)

        def consume(j, b, cnt):
            o = 1 - b
            pltpu.make_async_copy(kqt0.at[pl.ds(0, _CE)], kqbufs[o], gsem).wait()
            pltpu.make_async_copy(vt0.at[pl.ds(0, _CE)], obufs[o], gsem).wait()

            @pl.when(j == cnt)
            def _():
                pltpu.make_async_copy(
                    or0, acc_sh.at[pl.ds(0, _CE)], ssem).wait()
                pltpu.make_async_copy(
                    srows, sco_hbm.at[pl.ds(0, _CE)], wsem).wait()
            for t in range(_CE // 16):
                sl = pl.ds(t * 16, 16)
                dstc[sl] = ebufs[o][1, sl]
                connc[sl] = plsc.bitcast(ebufs[o][2, sl], jnp.float32)

            @pl.when(j + 1 < cnt)
            def _():
                nxt = start + j + 1
                pltpu.async_copy(edata.at[pl.ds(nxt * 4, 4)], ebufs[o], isem)

            def grp(g, _):
                ev = g * 16 + iota
                cn = connc[pl.ds(g * 16, 16)]
                for h in range(2):
                    sacc = zero16
                    for t in range(16):
                        rot = jnp.bitwise_and(t + iota, 15)
                        sacc = sacc + (
                            plsc.load_gather(kqbufs[o], [ev, h * 16 + rot])
                            * plsc.load_gather(kqbufs[o],
                                               [ev, 32 + h * 16 + rot]))
                    sc = jnp.clip(sacc * cn * scale, -5.0, 5.0)
                    sc = jnp.exp(sc)
                    for t in range(16):
                        rot = jnp.bitwise_and(t + iota, 15)
                        colv = h * 16 + rot
                        vv = plsc.load_gather(obufs[o], [ev, colv])
                        plsc.store_scatter(obufs[o], [ev, colv], vv * sc)
                    plsc.store_scatter(
                        srows, [ev, jnp.full((16,), h, jnp.int32)], sc)
                return 0
            lax.fori_loop(0, _CE // 16, grp, 0)
            pltpu.async_copy(obufs[o], acc_sh.at[dstc], ssem, add=True)
            pltpu.async_copy(
                srows, sco_hbm.at[pl.ds(c * E + (start + j - 1) * _CE, _CE)],
                wsem)

        pltpu.async_copy(edata.at[pl.ds(start * 4, 4)], ebufs[0], isem)
        pltpu.async_copy(edata.at[pl.ds((start + 1) * 4, 4)], ebufs[1], isem)

        def pair_body(jj, _):
            for b in range(2):
                j = jj * 2 + b

                @pl.when(j < cnt)
                def _():
                    fetch(j, b)

                @pl.when(jnp.logical_and(j >= 1, j <= cnt))
                def _():
                    consume(j, b, cnt)
            return 0
        lax.fori_loop(0, (391 + 2) // 2, pair_body, 0)

        pltpu.make_async_copy(or0, acc_sh.at[pl.ds(0, _CE)], ssem).wait()
        pltpu.make_async_copy(srows, sco_hbm.at[pl.ds(0, _CE)], wsem).wait()
        plsc.subcore_barrier()

        def cpout(i, _):
            cid = i * 16 + s

            @pl.when(cid < _NZF)
            def _():
                pltpu.sync_copy(acc_sh.at[pl.ds(cid * _CE, _CE)], or0)
                pltpu.sync_copy(or0, acc_hbm.at[pl.ds(cN + cid * _CE, _CE)])
            return 0
        lax.fori_loop(0, (_NZF + 15) // 16, cpout, 0)

        @pl.when(s == 0)
        def _():
            tail = N - _NZF * _CE
            pltpu.sync_copy(acc_sh.at[pl.ds(_NZF * _CE, tail)],
                            or0.at[pl.ds(0, tail)])
            pltpu.sync_copy(or0.at[pl.ds(0, tail)],
                            acc_hbm.at[pl.ds(cN + _NZF * _CE, tail)])

    return body


def _z_sc():
    mesh = plsc.VectorSubcoreMesh(core_axis_name="c", subcore_axis_name="s")

    @functools.partial(
        pl.kernel, mesh=mesh,
        compiler_params=pltpu.CompilerParams(use_tc_tiling_on_sc=False,
                                             needs_layout_passes=False),
        out_type=jax.ShapeDtypeStruct((2 * N, 8), jnp.float32),
        scratch_types=[
            pltpu.VMEM((_CE,), jnp.int32),
            pltpu.VMEM((_CE, 8), jnp.float32),
            pltpu.VMEM_SHARED((N, 8), jnp.float32),
        ],
    )
    def body(scol, dstl, z_hbm, dst_v, srows, accz):
        c = lax.axis_index("c")
        s = lax.axis_index("s")
        cN = c * N
        iota = lax.iota(jnp.int32, 16)
        zero16 = jnp.zeros((16,), jnp.float32)

        def zs(i, _):
            fl = i * 16 + iota
            plsc.store_scatter(srows, [fl // 8, fl % 8], zero16)
            return 0
        lax.fori_loop(0, _CE * 8 // 16, zs, 0)

        def zinit(i, _):
            cid = i * 16 + s

            @pl.when(cid < _NZF)
            def _():
                pltpu.sync_copy(srows, accz.at[pl.ds(cid * _CE, _CE)])
            return 0
        lax.fori_loop(0, (_NZF + 15) // 16, zinit, 0)

        @pl.when(s == 0)
        def _():
            pltpu.sync_copy(srows.at[pl.ds(0, N - _NZF * _CE)],
                            accz.at[pl.ds(_NZF * _CE, N - _NZF * _CE)])
        plsc.subcore_barrier()

        def chunk_body(i, _):
            cid = i * 16 + s

            @pl.when(cid < _NCHUNK)
            def _():
                base = cid * _CE
                pltpu.sync_copy(dstl.at[pl.ds(base, _CE)], dst_v)
                pltpu.sync_copy(scol.at[pl.ds(c * E + base, _CE)], srows)
                pltpu.sync_copy(srows, accz.at[dst_v], add=True)
            return 0
        lax.fori_loop(0, (_NCHUNK + 15) // 16, chunk_body, 0)
        plsc.subcore_barrier()

        def cpout(i, _):
            cid = i * 16 + s

            @pl.when(cid < _NZF)
            def _():
                pltpu.sync_copy(accz.at[pl.ds(cid * _CE, _CE)], srows)
                pltpu.sync_copy(srows, z_hbm.at[pl.ds(cN + cid * _CE, _CE)])
            return 0
        lax.fori_loop(0, (_NZF + 15) // 16, cpout, 0)

        @pl.when(s == 0)
        def _():
            tail = N - _NZF * _CE
            pltpu.sync_copy(accz.at[pl.ds(_NZF * _CE, tail)],
                            srows.at[pl.ds(0, tail)])
            pltpu.sync_copy(srows.at[pl.ds(0, tail)],
                            z_hbm.at[pl.ds(cN + _NZF * _CE, tail)])

    return body


def _pack_edata(src, dst, conn):
    connb = jax.lax.bitcast_convert_type(conn, jnp.int32)
    ed = jnp.stack([src, dst, connb, jnp.zeros_like(src)], 0)
    return ed.reshape(4, _NCHUNK, _CE).transpose(1, 0, 2).reshape(
        _NCHUNK * 4, _CE)


def _msg_sc(kq0, kq1, v0, v1, edata, dst, in_feats):
    fn = _make_edge_sc(float(1.0 / np.sqrt(in_feats)))
    acc, scores = fn(kq0, kq1, v0, v1, edata)
    zout = _z_sc()(scores, dst)
    wv = jnp.concatenate([acc[:N, :32], acc[N:, :32]], 1)
    z = jnp.concatenate([zout[:N, 0:2], zout[N:, 0:2]], 1)
    return wv, z


def _gather_sc(table, cs, cd):
    mesh = plsc.VectorSubcoreMesh(core_axis_name="c", subcore_axis_name="s")

    @functools.partial(
        pl.kernel, mesh=mesh,
        compiler_params=pltpu.CompilerParams(use_tc_tiling_on_sc=False,
                                             needs_layout_passes=False),
        out_type=[jax.ShapeDtypeStruct((E, 80), jnp.float32)] * 2,
        scratch_types=[
            pltpu.VMEM((_CE,), jnp.int32),
            pltpu.VMEM((_CE,), jnp.int32),
            pltpu.VMEM((_CE, 80), jnp.float32),
            pltpu.VMEM((_CE, 80), jnp.float32),
            pltpu.SemaphoreType.DMA,
        ],
    )
    def body(tab, csl, cdl, gcs_hbm, gcd_hbm, cs_v, cd_v, rows_a, rows_b, sem):
        c = lax.axis_index("c")
        s = lax.axis_index("s")
        w = s * 2 + c

        def chunk_body(i, _):
            cid = i * 32 + w

            @pl.when(cid < _NCHUNK)
            def _():
                base = cid * _CE
                pltpu.sync_copy(csl.at[pl.ds(base, _CE)], cs_v)
                pltpu.sync_copy(cdl.at[pl.ds(base, _CE)], cd_v)
                pltpu.async_copy(tab.at[cs_v], rows_a, sem).wait()
                pltpu.sync_copy(rows_a, gcs_hbm.at[pl.ds(base, _CE)])
                pltpu.async_copy(tab.at[cd_v], rows_b, sem).wait()
                pltpu.sync_copy(rows_b, gcd_hbm.at[pl.ds(base, _CE)])
            return 0
        lax.fori_loop(0, (_NCHUNK + 31) // 32, chunk_body, 0)

    return body(table, cs, cd)


def kernel(node_feat, coord, edge_feat, conn_feat, params, od_edge_index,
           connect_edge_index):
    p = params
    c8 = jnp.pad(coord, ((0, 0), (0, 6)))
    zeros86 = jnp.zeros((8, 64), jnp.float32)
    os_, od_ = od_edge_index[0], od_edge_index[1]
    cs_, cd_ = connect_edge_index[0], connect_edge_index[1]

    h0, kq0, kq1, v0, v1 = _node_dense(node_feat, c8, p)

    ones_e = jnp.ones((E,), jnp.float32)
    conn_e = conn_feat[:, 0]
    edata_od = _pack_edata(os_, od_, ones_e)
    edata_cc = _pack_edata(cs_, cd_, conn_e)

    wv, z = _msg_sc(kq0, kq1, v0, v1, edata_od, od_, 66.0)
    z8 = jnp.pad(z, ((0, 0), (0, 4)), constant_values=1.0)
    wf = p["conv1"]["f"]["W"]
    y1 = _combine(wv, z8, h0, c8, p["conv1"], wf[:64],
                  jnp.pad(wf[64:66], ((0, 6), (0, 0))), 1.0)

    kq0, kq1, v0, v1 = _qkv(y1, p["cc1"])
    wv, z = _msg_sc(kq0, kq1, v0, v1, edata_cc, cd_, 64.0)
    z8 = jnp.pad(z, ((0, 0), (0, 4)), constant_values=1.0)
    y2 = _combine(wv, z8, y1, c8, p["cc1"], p["cc1"]["f"]["W"], zeros86, 0.0)

    kq0, kq1, v0, v1 = _qkv(y2, p["cc2"])
    wv, z = _msg_sc(kq0, kq1, v0, v1, edata_cc, cd_, 64.0)
    z8 = jnp.pad(z, ((0, 0), (0, 4)), constant_values=1.0)
    y3 = _combine(wv, z8, y2, c8, p["cc2"], p["cc2"]["f"]["W"], zeros86, 0.0)

    hc80 = jnp.pad(jnp.concatenate([y3, coord], 1), ((0, 0), (0, 14)))
    gcs, gcd = _gather_sc(hc80, cs_, cd_)
    ef8 = jnp.pad(edge_feat, ((0, 0), (0, 6)))
    return _edge_mlp(gcs, gcd, ef8, p)
